# Initial kernel scaffold; baseline (speedup 1.0000x reference)
import jax
import jax.numpy as jnp
from jax.experimental import pallas as pl


def kernel(nfeats, edge_index, efeats, W_msg1, b_msg1, W_apply1, b_apply1, W_msg2, b_msg2, W_apply2, b_apply2, W_pred, b_pred):
    raise NotImplementedError("write your pallas kernel here")


# speedup vs baseline: 5.0514x; 5.0514x over previous
import functools

import jax
import jax.numpy as jnp
from jax import lax
from jax.experimental import pallas as pl
from jax.experimental.pallas import tpu as pltpu
from jax.experimental.pallas import tpu_sc as plsc

N = 10000
E = 160000
DIN = 256
DE = 16
DH = 128
DOUT = 256

NC = 2
NS = 16
NW = NC * NS
EPT = E // NW
CH = 125
NCH = EPT // CH
STRIPE = 640
RZC = 80

_mesh = plsc.VectorSubcoreMesh(core_axis_name="c", subcore_axis_name="s")
_f32 = jnp.float32


def _zero_vmem_2d(ref, nrows, ncols):
    zeros16 = jnp.zeros((16,), _f32)

    def body(r, carry):
        for cc in range(ncols // 16):
            ref[r, pl.ds(cc * 16, 16)] = zeros16
        return carry

    lax.fori_loop(0, nrows, body, None)


def _ef_body(ef_hbm, dst3_hbm, out_a_hbm, acc_a, dbuf, efbuf):
    c = lax.axis_index("c")
    s = lax.axis_index("s")
    wid = c * NS + s
    pltpu.sync_copy(dst3_hbm.at[wid], dbuf)
    _zero_vmem_2d(efbuf, RZC, 32)
    r0 = s * STRIPE
    nz = jnp.where(s < NS - 1, STRIPE // RZC, (N - (NS - 1) * STRIPE) // RZC)

    def zc(q, carry):
        pltpu.sync_copy(efbuf.at[pl.ds(0, RZC)],
                        acc_a.at[pl.ds(r0 + q * RZC, RZC)])
        return carry

    lax.fori_loop(0, nz, zc, None)
    plsc.subcore_barrier()

    def body(k, carry):
        pltpu.sync_copy(ef_hbm.at[wid].at[k], efbuf)
        pltpu.sync_copy(efbuf, acc_a.at[dbuf.at[k]], add=True)
        return carry

    lax.fori_loop(0, NCH, body, None)
    plsc.subcore_barrier()

    def oc(q, carry):
        pltpu.sync_copy(acc_a.at[pl.ds(r0 + q * RZC, RZC)],
                        out_a_hbm.at[c].at[pl.ds(r0 + q * RZC, RZC)])
        return carry

    lax.fori_loop(0, nz, oc, None)


def _gs_body(table_hbm, src3_hbm, dst3_hbm, out_b_hbm,
             acc_b, sbuf, dbuf, rows, sem):
    c = lax.axis_index("c")
    s = lax.axis_index("s")
    wid = c * NS + s
    pltpu.sync_copy(src3_hbm.at[wid], sbuf)
    pltpu.sync_copy(dst3_hbm.at[wid], dbuf)
    _zero_vmem_2d(rows, RZC, DH)
    r0 = s * STRIPE
    nz = jnp.where(s < NS - 1, STRIPE // RZC, (N - (NS - 1) * STRIPE) // RZC)

    def zc(q, carry):
        pltpu.sync_copy(rows.at[pl.ds(0, RZC)],
                        acc_b.at[pl.ds(r0 + q * RZC, RZC)])
        return carry

    lax.fori_loop(0, nz, zc, None)
    plsc.subcore_barrier()

    def body(k, carry):
        pltpu.async_copy(table_hbm.at[sbuf.at[k]], rows, sem).wait()
        pltpu.sync_copy(rows, acc_b.at[dbuf.at[k]], add=True)
        return carry

    lax.fori_loop(0, NCH, body, None)
    plsc.subcore_barrier()

    def oc(q, carry):
        pltpu.sync_copy(acc_b.at[pl.ds(r0 + q * RZC, RZC)],
                        out_b_hbm.at[c].at[pl.ds(r0 + q * RZC, RZC)])
        return carry

    lax.fori_loop(0, nz, oc, None)


_ef_call = functools.partial(
    pl.kernel,
    out_type=jax.ShapeDtypeStruct((NC, N, 32), _f32),
    mesh=_mesh,
    compiler_params=pltpu.CompilerParams(use_tc_tiling_on_sc=False),
    scratch_types=[
        pltpu.VMEM_SHARED((N, 32), _f32),
        pltpu.VMEM((NCH, CH), jnp.int32),
        pltpu.VMEM((CH, 32), _f32),
    ],
)(_ef_body)

_gs_call = functools.partial(
    pl.kernel,
    out_type=jax.ShapeDtypeStruct((NC, N, DH), _f32),
    mesh=_mesh,
    compiler_params=pltpu.CompilerParams(use_tc_tiling_on_sc=False),
    scratch_types=[
        pltpu.VMEM_SHARED((N, DH), _f32),
        pltpu.VMEM((NCH, CH), jnp.int32),
        pltpu.VMEM((NCH, CH), jnp.int32),
        pltpu.VMEM((CH, DH), _f32),
        pltpu.SemaphoreType.DMA,
    ],
)(_gs_body)


def _pred_body(uv_hbm, src_hbm, dst_hbm, ep_hbm, out_hbm,
               uvbuf, sbuf, dbuf, epbuf, outbuf):
    c = lax.axis_index("c")
    s = lax.axis_index("s")
    wid = c * NS + s
    base = wid * EPT
    pltpu.sync_copy(uv_hbm, uvbuf)
    pltpu.sync_copy(src_hbm.at[pl.ds(base, EPT)], sbuf.at[pl.ds(0, EPT)])
    pltpu.sync_copy(dst_hbm.at[pl.ds(base, EPT)], dbuf.at[pl.ds(0, EPT)])
    pltpu.sync_copy(ep_hbm.at[pl.ds(base * 2, EPT * 2)], epbuf)
    iota = lax.iota(jnp.int32, 16)
    nfull = EPT // 16

    def step(j, mask):
        sv = sbuf[pl.ds(j * 16, 16)] * 4
        dv = dbuf[pl.ds(j * 16, 16)] * 4
        u0 = plsc.load_gather(uvbuf, [sv], mask=mask)
        u1 = plsc.load_gather(uvbuf, [sv + 1], mask=mask)
        v0 = plsc.load_gather(uvbuf, [dv + 2], mask=mask)
        v1 = plsc.load_gather(uvbuf, [dv + 3], mask=mask)
        o = j * 32 + 2 * iota
        e0 = plsc.load_gather(epbuf, [o], mask=mask)
        e1 = plsc.load_gather(epbuf, [o + 1], mask=mask)
        plsc.store_scatter(outbuf, [o], u0 + v0 + e0, mask=mask)
        plsc.store_scatter(outbuf, [o + 1], u1 + v1 + e1, mask=mask)

    def body(j, carry):
        step(j, None)
        return carry

    lax.fori_loop(0, nfull, body, None)
    step(nfull, iota < (EPT - nfull * 16))
    pltpu.sync_copy(outbuf, out_hbm.at[pl.ds(base * 2, EPT * 2)])


_pred_call = functools.partial(
    pl.kernel,
    out_type=jax.ShapeDtypeStruct((2 * E,), _f32),
    mesh=_mesh,
    compiler_params=pltpu.CompilerParams(needs_layout_passes=False, use_tc_tiling_on_sc=False),
    scratch_types=[
        pltpu.VMEM((N * 4,), _f32),
        pltpu.VMEM((EPT + 16, ), jnp.int32),
        pltpu.VMEM((EPT + 16, ), jnp.int32),
        pltpu.VMEM((2 * EPT,), _f32),
        pltpu.VMEM((2 * EPT,), _f32),
    ],
)(_pred_body)


_NB = 1000
_EB = 8000


def _mm_body(x_ref, w_ref, o_ref):
    o_ref[...] = jnp.dot(x_ref[...], w_ref[...],
                         preferred_element_type=_f32)


def _tc_matmul(x, w):
    n, k = x.shape
    m = w.shape[1]
    return pl.pallas_call(
        _mm_body,
        grid=(n // _NB,),
        in_specs=[pl.BlockSpec((_NB, k), lambda i: (i, 0)),
                  pl.BlockSpec((k, m), lambda i: (0, 0))],
        out_specs=pl.BlockSpec((_NB, m), lambda i: (i, 0)),
        out_shape=jax.ShapeDtypeStruct((n, m), _f32),
    )(x, w)


def _apply_body(h_ref, pb_ref, pa_ref, wmsg_ref, we_ref, b_ref,
                wah_ref, wan_ref, ba_ref, o_ref):
    pa = pa_ref[0] + pa_ref[1]
    ge = pa[:, :DE]
    deg = pa[:, DE]
    gs = pb_ref[0] + pb_ref[1]
    if wmsg_ref is not None:
        gs = jnp.dot(gs, wmsg_ref[...], preferred_element_type=_f32)
    ssum = (gs + jnp.dot(ge, we_ref[...], preferred_element_type=_f32)
            + deg[:, None] * b_ref[...])
    hn = ssum / jnp.maximum(deg, 1.0)[:, None]
    o_ref[...] = jnp.maximum(
        jnp.dot(h_ref[...], wah_ref[...], preferred_element_type=_f32)
        + jnp.dot(hn, wan_ref[...], preferred_element_type=_f32)
        + ba_ref[...], 0.0)


def _tc_apply(h, pb, pa, wmsg, we, b, wah, wan, ba, dout):
    din = h.shape[1]
    dmid = pb.shape[2]
    have_wmsg = wmsg is not None
    body = _apply_body if have_wmsg else (
        lambda h_ref, pb_ref, pa_ref, we_ref, b_ref, wah_ref, wan_ref,
               ba_ref, o_ref:
        _apply_body(h_ref, pb_ref, pa_ref, None, we_ref, b_ref,
                    wah_ref, wan_ref, ba_ref, o_ref))
    in_specs = [pl.BlockSpec((_NB, din), lambda i: (i, 0)),
                pl.BlockSpec((NC, _NB, dmid), lambda i: (0, i, 0)),
                pl.BlockSpec((NC, _NB, 32), lambda i: (0, i, 0))]
    args = [h, pb, pa]
    if have_wmsg:
        in_specs.append(pl.BlockSpec(wmsg.shape, lambda i: (0, 0)))
        args.append(wmsg)
    in_specs += [pl.BlockSpec(we.shape, lambda i: (0, 0)),
                 pl.BlockSpec((1, dout), lambda i: (0, 0)),
                 pl.BlockSpec(wah.shape, lambda i: (0, 0)),
                 pl.BlockSpec(wan.shape, lambda i: (0, 0)),
                 pl.BlockSpec((1, dout), lambda i: (0, 0))]
    args += [we, b.reshape(1, dout), wah, wan, ba.reshape(1, dout)]
    return pl.pallas_call(
        body,
        grid=(N // _NB,),
        in_specs=in_specs,
        out_specs=pl.BlockSpec((_NB, dout), lambda i: (i, 0)),
        out_shape=jax.ShapeDtypeStruct((N, dout), _f32),
    )(*args)


def _uv_body(h_ref, pb_ref, pa_ref, wmsg_ref, we_ref, b_ref,
             wah_ref, wan_ref, ba_ref, wuv_ref, o_ref):
    pa = pa_ref[0] + pa_ref[1]
    ge = pa[:, :DE]
    deg = pa[:, DE]
    gs = jnp.dot(pb_ref[0] + pb_ref[1], wmsg_ref[...],
                 preferred_element_type=_f32)
    ssum = (gs + jnp.dot(ge, we_ref[...], preferred_element_type=_f32)
            + deg[:, None] * b_ref[...])
    hn = ssum / jnp.maximum(deg, 1.0)[:, None]
    h2 = jnp.maximum(
        jnp.dot(h_ref[...], wah_ref[...], preferred_element_type=_f32)
        + jnp.dot(hn, wan_ref[...], preferred_element_type=_f32)
        + ba_ref[...], 0.0)
    o_ref[...] = jnp.dot(h2, wuv_ref[...], preferred_element_type=_f32)


def _ep_body(ef_ref, w_ref, b_ref, o_ref):
    o_ref[...] = jnp.dot(ef_ref[...], w_ref[...],
                         preferred_element_type=_f32) + b_ref[...]


def kernel(nfeats, edge_index, efeats,
           W_msg1, b_msg1, W_apply1, b_apply1,
           W_msg2, b_msg2, W_apply2, b_apply2,
           W_pred, b_pred):
    h0 = nfeats.reshape(N, DIN)
    ef2 = efeats.reshape(E, DE)
    ei = edge_index.astype(jnp.int32)
    src = ei[0]
    dst = ei[1]
    e3 = ei.reshape(2, NW, NCH, CH)
    src3 = e3[0]
    dst3 = e3[1]
    ef_aug = jnp.concatenate(
        [ef2, jnp.ones((E, 1), _f32), jnp.zeros((E, 15), _f32)], axis=1)

    Wm1h, We1 = W_msg1[:DIN], W_msg1[DIN:]
    Wa1h, Wa1n = W_apply1[:DIN], W_apply1[DIN:]
    Wm2h, We2 = W_msg2[:DH], W_msg2[DH:]
    Wa2h, Wa2n = W_apply2[:DH], W_apply2[DH:]
    Wuv = jnp.concatenate([W_pred[:DOUT], W_pred[DOUT:2 * DOUT]], axis=1)
    Wpe = W_pred[2 * DOUT:]

    t1 = _tc_matmul(h0, Wm1h)

    ep = pl.pallas_call(
        _ep_body,
        grid=(E // _EB,),
        in_specs=[pl.BlockSpec((_EB, DE), lambda i: (i, 0)),
                  pl.BlockSpec((DE, 2), lambda i: (0, 0)),
                  pl.BlockSpec((1, 2), lambda i: (0, 0))],
        out_specs=pl.BlockSpec((_EB, 2), lambda i: (i, 0)),
        out_shape=jax.ShapeDtypeStruct((E, 2), _f32),
    )(ef2, Wpe, b_pred.reshape(1, 2))

    pa = _ef_call(ef_aug.reshape(NW, NCH, CH, 32), dst3)
    pb1 = _gs_call(t1, src3, dst3)

    h1 = _tc_apply(h0, pb1, pa, None, We1, b_msg1, Wa1h, Wa1n, b_apply1, DH)

    pb2 = _gs_call(h1, src3, dst3)

    uv = pl.pallas_call(
        _uv_body,
        grid=(N // _NB,),
        in_specs=[pl.BlockSpec((_NB, DH), lambda i: (i, 0)),
                  pl.BlockSpec((NC, _NB, DH), lambda i: (0, i, 0)),
                  pl.BlockSpec((NC, _NB, 32), lambda i: (0, i, 0)),
                  pl.BlockSpec((DH, DOUT), lambda i: (0, 0)),
                  pl.BlockSpec((DE, DOUT), lambda i: (0, 0)),
                  pl.BlockSpec((1, DOUT), lambda i: (0, 0)),
                  pl.BlockSpec((DH, DOUT), lambda i: (0, 0)),
                  pl.BlockSpec((DOUT, DOUT), lambda i: (0, 0)),
                  pl.BlockSpec((1, DOUT), lambda i: (0, 0)),
                  pl.BlockSpec((DOUT, 4), lambda i: (0, 0))],
        out_specs=pl.BlockSpec((_NB, 4), lambda i: (i, 0)),
        out_shape=jax.ShapeDtypeStruct((N, 4), _f32),
    )(h1, pb2, pa, Wm2h, We2, b_msg2.reshape(1, DOUT),
      Wa2h, Wa2n, b_apply2.reshape(1, DOUT), Wuv)

    score_flat = _pred_call(uv.reshape(N * 4), src, dst, ep.reshape(2 * E))
    return score_flat.reshape(E, 2)


# speedup vs baseline: 5.5414x; 1.0970x over previous
import functools

import jax
import jax.numpy as jnp
from jax import lax
from jax.experimental import pallas as pl
from jax.experimental.pallas import tpu as pltpu
from jax.experimental.pallas import tpu_sc as plsc

N = 10000
E = 160000
DIN = 256
DE = 16
DH = 128
DOUT = 256

NC = 2
NS = 16
NW = NC * NS
EPT = E // NW
CH = 100
NCH = EPT // CH
STRIPE = 640
RZC = 80

_mesh = plsc.VectorSubcoreMesh(core_axis_name="c", subcore_axis_name="s")
_f32 = jnp.float32


def _zero_vmem_2d(ref, nrows, ncols):
    zeros16 = jnp.zeros((16,), _f32)

    def body(r, carry):
        for cc in range(ncols // 16):
            ref[r, pl.ds(cc * 16, 16)] = zeros16
        return carry

    lax.fori_loop(0, nrows, body, None)


def _ef_body(ef_hbm, dst3_hbm, out_a_hbm, acc_a, dbuf, efbuf):
    c = lax.axis_index("c")
    s = lax.axis_index("s")
    wid = c * NS + s
    pltpu.sync_copy(dst3_hbm.at[wid], dbuf)
    _zero_vmem_2d(efbuf, CH, 32)
    r0 = s * STRIPE
    nz = jnp.where(s < NS - 1, STRIPE // RZC, (N - (NS - 1) * STRIPE) // RZC)

    def zc(q, carry):
        pltpu.sync_copy(efbuf.at[pl.ds(0, RZC)],
                        acc_a.at[pl.ds(r0 + q * RZC, RZC)])
        return carry

    lax.fori_loop(0, nz, zc, None)

    ones16 = jnp.where(lax.iota(jnp.int32, 16) == 0,
                       jnp.float32(1.0), jnp.float32(0.0))

    def setone(r, carry):
        efbuf[r, pl.ds(DE, 16)] = ones16
        return carry

    lax.fori_loop(0, CH, setone, None)
    plsc.subcore_barrier()

    def body(k, carry):
        pltpu.sync_copy(ef_hbm.at[wid].at[k], efbuf.at[:, pl.ds(0, DE)])
        pltpu.sync_copy(efbuf, acc_a.at[dbuf.at[k]], add=True)
        return carry

    lax.fori_loop(0, NCH, body, None)
    plsc.subcore_barrier()

    def oc(q, carry):
        pltpu.sync_copy(acc_a.at[pl.ds(r0 + q * RZC, RZC)],
                        out_a_hbm.at[c].at[pl.ds(r0 + q * RZC, RZC)])
        return carry

    lax.fori_loop(0, nz, oc, None)


def _gs_body(table_hbm, src3_hbm, dst3_hbm, out_b_hbm,
             acc_b, sbuf, dbuf, rows0, rows1, sem0, sem1):
    c = lax.axis_index("c")
    s = lax.axis_index("s")
    wid = c * NS + s
    pltpu.sync_copy(src3_hbm.at[wid], sbuf)
    pltpu.sync_copy(dst3_hbm.at[wid], dbuf)
    _zero_vmem_2d(rows0, RZC, DH)
    r0 = s * STRIPE
    nz = jnp.where(s < NS - 1, STRIPE // RZC, (N - (NS - 1) * STRIPE) // RZC)

    def zc(q, carry):
        pltpu.sync_copy(rows0.at[pl.ds(0, RZC)],
                        acc_b.at[pl.ds(r0 + q * RZC, RZC)])
        return carry

    lax.fori_loop(0, nz, zc, None)
    plsc.subcore_barrier()

    pltpu.async_copy(table_hbm.at[sbuf.at[0]], rows0, sem0)

    def body(p, carry):
        k = p * 2
        pltpu.async_copy(table_hbm.at[sbuf.at[k + 1]], rows1, sem1)
        pltpu.make_async_copy(table_hbm.at[sbuf.at[k]], rows0, sem0).wait()
        pltpu.sync_copy(rows0, acc_b.at[dbuf.at[k]], add=True)

        @pl.when(k + 2 < NCH)
        def _():
            pltpu.async_copy(table_hbm.at[sbuf.at[k + 2]], rows0, sem0)

        pltpu.make_async_copy(table_hbm.at[sbuf.at[k + 1]], rows1, sem1).wait()
        pltpu.sync_copy(rows1, acc_b.at[dbuf.at[k + 1]], add=True)
        return carry

    lax.fori_loop(0, NCH // 2, body, None)
    plsc.subcore_barrier()

    def oc(q, carry):
        pltpu.sync_copy(acc_b.at[pl.ds(r0 + q * RZC, RZC)],
                        out_b_hbm.at[c].at[pl.ds(r0 + q * RZC, RZC)])
        return carry

    lax.fori_loop(0, nz, oc, None)


_ef_call = functools.partial(
    pl.kernel,
    out_type=jax.ShapeDtypeStruct((NC, N, 32), _f32),
    mesh=_mesh,
    compiler_params=pltpu.CompilerParams(use_tc_tiling_on_sc=False),
    scratch_types=[
        pltpu.VMEM_SHARED((N, 32), _f32),
        pltpu.VMEM((NCH, CH), jnp.int32),
        pltpu.VMEM((CH, 32), _f32),
    ],
)(_ef_body)

_gs_call = functools.partial(
    pl.kernel,
    out_type=jax.ShapeDtypeStruct((NC, N, DH), _f32),
    mesh=_mesh,
    compiler_params=pltpu.CompilerParams(use_tc_tiling_on_sc=False),
    scratch_types=[
        pltpu.VMEM_SHARED((N, DH), _f32),
        pltpu.VMEM((NCH, CH), jnp.int32),
        pltpu.VMEM((NCH, CH), jnp.int32),
        pltpu.VMEM((CH, DH), _f32),
        pltpu.VMEM((CH, DH), _f32),
        pltpu.SemaphoreType.DMA,
        pltpu.SemaphoreType.DMA,
    ],
)(_gs_body)


def _pred_body(uv_hbm, src_hbm, dst_hbm, ep_hbm, out_hbm,
               uvbuf, sbuf, dbuf, epbuf, outbuf):
    c = lax.axis_index("c")
    s = lax.axis_index("s")
    wid = c * NS + s
    base = wid * EPT
    pltpu.sync_copy(uv_hbm, uvbuf)
    pltpu.sync_copy(src_hbm.at[pl.ds(base, EPT)], sbuf.at[pl.ds(0, EPT)])
    pltpu.sync_copy(dst_hbm.at[pl.ds(base, EPT)], dbuf.at[pl.ds(0, EPT)])
    pltpu.sync_copy(ep_hbm.at[pl.ds(base * 2, EPT * 2)], epbuf)
    iota = lax.iota(jnp.int32, 16)
    nfull = EPT // 16

    def step(j, mask):
        sv = sbuf[pl.ds(j * 16, 16)] * 4
        dv = dbuf[pl.ds(j * 16, 16)] * 4
        u0 = plsc.load_gather(uvbuf, [sv], mask=mask)
        u1 = plsc.load_gather(uvbuf, [sv + 1], mask=mask)
        v0 = plsc.load_gather(uvbuf, [dv + 2], mask=mask)
        v1 = plsc.load_gather(uvbuf, [dv + 3], mask=mask)
        o = j * 32 + 2 * iota
        e0 = plsc.load_gather(epbuf, [o], mask=mask)
        e1 = plsc.load_gather(epbuf, [o + 1], mask=mask)
        plsc.store_scatter(outbuf, [o], u0 + v0 + e0, mask=mask)
        plsc.store_scatter(outbuf, [o + 1], u1 + v1 + e1, mask=mask)

    def body(j, carry):
        step(j, None)
        return carry

    lax.fori_loop(0, nfull, body, None)
    step(nfull, iota < (EPT - nfull * 16))
    pltpu.sync_copy(outbuf, out_hbm.at[pl.ds(base * 2, EPT * 2)])


_pred_call = functools.partial(
    pl.kernel,
    out_type=jax.ShapeDtypeStruct((2 * E,), _f32),
    mesh=_mesh,
    compiler_params=pltpu.CompilerParams(needs_layout_passes=False, use_tc_tiling_on_sc=False),
    scratch_types=[
        pltpu.VMEM((N * 4,), _f32),
        pltpu.VMEM((EPT + 16, ), jnp.int32),
        pltpu.VMEM((EPT + 16, ), jnp.int32),
        pltpu.VMEM((2 * EPT,), _f32),
        pltpu.VMEM((2 * EPT,), _f32),
    ],
)(_pred_body)


_NB = 1000
_EB = 8000


def _mm_body(x_ref, w_ref, o_ref):
    o_ref[...] = jnp.dot(x_ref[...], w_ref[...],
                         preferred_element_type=_f32)


def _tc_matmul(x, w):
    n, k = x.shape
    m = w.shape[1]
    return pl.pallas_call(
        _mm_body,
        grid=(n // _NB,),
        in_specs=[pl.BlockSpec((_NB, k), lambda i: (i, 0)),
                  pl.BlockSpec((k, m), lambda i: (0, 0))],
        out_specs=pl.BlockSpec((_NB, m), lambda i: (i, 0)),
        out_shape=jax.ShapeDtypeStruct((n, m), _f32),
    )(x, w)


def _apply_body(h_ref, pb_ref, pa_ref, wmsg_ref, we_ref, b_ref,
                wah_ref, wan_ref, ba_ref, o_ref):
    pa = pa_ref[0] + pa_ref[1]
    ge = pa[:, :DE]
    deg = pa[:, DE]
    gs = pb_ref[0] + pb_ref[1]
    if wmsg_ref is not None:
        gs = jnp.dot(gs, wmsg_ref[...], preferred_element_type=_f32)
    ssum = (gs + jnp.dot(ge, we_ref[...], preferred_element_type=_f32)
            + deg[:, None] * b_ref[...])
    hn = ssum / jnp.maximum(deg, 1.0)[:, None]
    o_ref[...] = jnp.maximum(
        jnp.dot(h_ref[...], wah_ref[...], preferred_element_type=_f32)
        + jnp.dot(hn, wan_ref[...], preferred_element_type=_f32)
        + ba_ref[...], 0.0)


def _tc_apply(h, pb, pa, wmsg, we, b, wah, wan, ba, dout):
    din = h.shape[1]
    dmid = pb.shape[2]
    have_wmsg = wmsg is not None
    body = _apply_body if have_wmsg else (
        lambda h_ref, pb_ref, pa_ref, we_ref, b_ref, wah_ref, wan_ref,
               ba_ref, o_ref:
        _apply_body(h_ref, pb_ref, pa_ref, None, we_ref, b_ref,
                    wah_ref, wan_ref, ba_ref, o_ref))
    in_specs = [pl.BlockSpec((_NB, din), lambda i: (i, 0)),
                pl.BlockSpec((NC, _NB, dmid), lambda i: (0, i, 0)),
                pl.BlockSpec((NC, _NB, 32), lambda i: (0, i, 0))]
    args = [h, pb, pa]
    if have_wmsg:
        in_specs.append(pl.BlockSpec(wmsg.shape, lambda i: (0, 0)))
        args.append(wmsg)
    in_specs += [pl.BlockSpec(we.shape, lambda i: (0, 0)),
                 pl.BlockSpec((1, dout), lambda i: (0, 0)),
                 pl.BlockSpec(wah.shape, lambda i: (0, 0)),
                 pl.BlockSpec(wan.shape, lambda i: (0, 0)),
                 pl.BlockSpec((1, dout), lambda i: (0, 0))]
    args += [we, b.reshape(1, dout), wah, wan, ba.reshape(1, dout)]
    return pl.pallas_call(
        body,
        grid=(N // _NB,),
        in_specs=in_specs,
        out_specs=pl.BlockSpec((_NB, dout), lambda i: (i, 0)),
        out_shape=jax.ShapeDtypeStruct((N, dout), _f32),
    )(*args)


def _uv_body(h_ref, pb_ref, pa_ref, wmsg_ref, we_ref, b_ref,
             wah_ref, wan_ref, ba_ref, wuv_ref, o_ref):
    pa = pa_ref[0] + pa_ref[1]
    ge = pa[:, :DE]
    deg = pa[:, DE]
    gs = jnp.dot(pb_ref[0] + pb_ref[1], wmsg_ref[...],
                 preferred_element_type=_f32)
    ssum = (gs + jnp.dot(ge, we_ref[...], preferred_element_type=_f32)
            + deg[:, None] * b_ref[...])
    hn = ssum / jnp.maximum(deg, 1.0)[:, None]
    h2 = jnp.maximum(
        jnp.dot(h_ref[...], wah_ref[...], preferred_element_type=_f32)
        + jnp.dot(hn, wan_ref[...], preferred_element_type=_f32)
        + ba_ref[...], 0.0)
    o_ref[...] = jnp.dot(h2, wuv_ref[...], preferred_element_type=_f32)


def _ep_body(ef_ref, w_ref, b_ref, o_ref):
    o_ref[...] = jnp.dot(ef_ref[...], w_ref[...],
                         preferred_element_type=_f32) + b_ref[...]


def kernel(nfeats, edge_index, efeats,
           W_msg1, b_msg1, W_apply1, b_apply1,
           W_msg2, b_msg2, W_apply2, b_apply2,
           W_pred, b_pred):
    h0 = nfeats.reshape(N, DIN)
    ef2 = efeats.reshape(E, DE)
    ei = edge_index.astype(jnp.int32)
    src = ei[0]
    dst = ei[1]
    e3 = ei.reshape(2, NW, NCH, CH)
    src3 = e3[0]
    dst3 = e3[1]
    Wm1h, We1 = W_msg1[:DIN], W_msg1[DIN:]
    Wa1h, Wa1n = W_apply1[:DIN], W_apply1[DIN:]
    Wm2h, We2 = W_msg2[:DH], W_msg2[DH:]
    Wa2h, Wa2n = W_apply2[:DH], W_apply2[DH:]
    Wuv = jnp.concatenate([W_pred[:DOUT], W_pred[DOUT:2 * DOUT]], axis=1)
    Wpe = W_pred[2 * DOUT:]

    t1 = _tc_matmul(h0, Wm1h)

    ep = pl.pallas_call(
        _ep_body,
        grid=(E // _EB,),
        in_specs=[pl.BlockSpec((_EB, DE), lambda i: (i, 0)),
                  pl.BlockSpec((DE, 2), lambda i: (0, 0)),
                  pl.BlockSpec((1, 2), lambda i: (0, 0))],
        out_specs=pl.BlockSpec((_EB, 2), lambda i: (i, 0)),
        out_shape=jax.ShapeDtypeStruct((E, 2), _f32),
    )(ef2, Wpe, b_pred.reshape(1, 2))

    pa = _ef_call(ef2.reshape(NW, NCH, CH, DE), dst3)
    pb1 = _gs_call(t1, src3, dst3)

    h1 = _tc_apply(h0, pb1, pa, None, We1, b_msg1, Wa1h, Wa1n, b_apply1, DH)

    pb2 = _gs_call(h1, src3, dst3)

    uv = pl.pallas_call(
        _uv_body,
        grid=(N // _NB,),
        in_specs=[pl.BlockSpec((_NB, DH), lambda i: (i, 0)),
                  pl.BlockSpec((NC, _NB, DH), lambda i: (0, i, 0)),
                  pl.BlockSpec((NC, _NB, 32), lambda i: (0, i, 0)),
                  pl.BlockSpec((DH, DOUT), lambda i: (0, 0)),
                  pl.BlockSpec((DE, DOUT), lambda i: (0, 0)),
                  pl.BlockSpec((1, DOUT), lambda i: (0, 0)),
                  pl.BlockSpec((DH, DOUT), lambda i: (0, 0)),
                  pl.BlockSpec((DOUT, DOUT), lambda i: (0, 0)),
                  pl.BlockSpec((1, DOUT), lambda i: (0, 0)),
                  pl.BlockSpec((DOUT, 4), lambda i: (0, 0))],
        out_specs=pl.BlockSpec((_NB, 4), lambda i: (i, 0)),
        out_shape=jax.ShapeDtypeStruct((N, 4), _f32),
    )(h1, pb2, pa, Wm2h, We2, b_msg2.reshape(1, DOUT),
      Wa2h, Wa2n, b_apply2.reshape(1, DOUT), Wuv)

    score_flat = _pred_call(uv.reshape(N * 4), src, dst, ep.reshape(2 * E))
    return score_flat.reshape(E, 2)
